# Initial kernel scaffold; baseline (speedup 1.0000x reference)
import jax
import jax.numpy as jnp
from jax.experimental import pallas as pl


def kernel(inputs, probs, training):
    raise NotImplementedError("write your pallas kernel here")


# speedup vs baseline: 4.8010x; 4.8010x over previous
import jax
import jax.numpy as jnp
from jax import lax
from jax.experimental import pallas as pl
from jax.experimental.pallas import tpu as pltpu
from jax.experimental.pallas import tpu_sc as plsc

_BATCH = 128
_N = 32768
_L = 16
_NSTEP = _N // _L
_NW = 32
_ROWS_PER_W = _BATCH // _NW


def _f32_to_key(x):
    i = plsc.bitcast(x, jnp.int32)
    u = plsc.bitcast(x, jnp.uint32)
    s = lax.shift_right_arithmetic(i, 31)
    flip = plsc.bitcast(s, jnp.uint32) | jnp.uint32(0x80000000)
    return u ^ flip


def _key_to_f32(k):
    neg = k < jnp.uint32(0x80000000)
    u = jnp.where(neg, ~k, k ^ jnp.uint32(0x80000000))
    return plsc.bitcast(u, jnp.float32)


def _sc_body(x_hbm, probs_hbm, out_hbm, row_v, hist_v, probs_v):
    c = lax.axis_index("c")
    s = lax.axis_index("s")
    wid = s * 2 + c
    pltpu.sync_copy(probs_hbm, probs_v)
    ones = jnp.ones((_L,), jnp.int32)

    def row_body(j, _):
        row = wid * _ROWS_PER_W + j
        pltpu.sync_copy(x_hbm.at[row], row_v)

        p16 = probs_v[pl.ds((row // _L) * _L, _L)]
        sel = lax.iota(jnp.int32, _L) == (row % _L)
        p = jnp.sum(jnp.where(sel, p16, 0.0))
        v = jnp.float32(_N) * p
        vi = v.astype(jnp.int32)
        ceil_v = vi + jnp.where(vi.astype(jnp.float32) < v, 1, 0)
        kv = jnp.clip(ceil_v - 1, 0, _N - 1)

        prefix = jnp.zeros((_L,), jnp.uint32)
        for level in range(4):
            shift = 24 - 8 * level
            for i in range(16):
                hist_v[pl.ds(i * _L, _L)] = jnp.zeros((_L,), jnp.int32)

            def hist_step(i, _, level=level, shift=shift, prefix=prefix):
                x = row_v[pl.ds(i * _L, _L)]
                key = _f32_to_key(x)
                bucket = (key >> jnp.uint32(shift)) & jnp.uint32(0xFF)
                if level == 0:
                    plsc.addupdate_scatter(
                        hist_v, [bucket.astype(jnp.int32)], ones)
                else:
                    m = (key >> jnp.uint32(shift + 8)) == prefix
                    plsc.addupdate_scatter(
                        hist_v, [bucket.astype(jnp.int32)], ones, mask=m)
                return 0

            lax.fori_loop(0, _NSTEP, hist_step, 0)

            carry = jnp.zeros((_L,), jnp.int32)
            bacc = jnp.zeros((_L,), jnp.int32)
            wacc = jnp.zeros((_L,), jnp.int32)
            for i in range(16):
                h = hist_v[pl.ds(i * _L, _L)]
                run = carry + plsc.cumsum(h)
                le = run <= kv
                bacc = bacc + jnp.where(le, 1, 0)
                wacc = wacc + jnp.where(le, h, 0)
                carry = carry + jnp.sum(h)
            bucket = jnp.sum(bacc)
            below = jnp.sum(wacc)
            kv = kv - below
            prefix = (prefix << jnp.uint32(8)) | bucket.astype(jnp.uint32)

        thresh = _key_to_f32(prefix)

        def mask_step(i, _, thresh=thresh):
            x = row_v[pl.ds(i * _L, _L)]
            row_v[pl.ds(i * _L, _L)] = jnp.where(x >= thresh, x, 0.0)
            return 0

        lax.fori_loop(0, _NSTEP, mask_step, 0)
        pltpu.sync_copy(row_v, out_hbm.at[row])
        return 0

    lax.fori_loop(0, _ROWS_PER_W, row_body, 0)


_sc_masking = pl.kernel(
    _sc_body,
    out_type=jax.ShapeDtypeStruct((_BATCH, _N), jnp.float32),
    mesh=plsc.VectorSubcoreMesh(core_axis_name="c", subcore_axis_name="s"),
    compiler_params=pltpu.CompilerParams(needs_layout_passes=False),
    scratch_types=[
        pltpu.VMEM((_N,), jnp.float32),
        pltpu.VMEM((256,), jnp.int32),
        pltpu.VMEM((_BATCH,), jnp.float32),
    ],
)


@jax.jit
def _masked(inputs, probs):
    return _sc_masking(inputs, probs)


def kernel(inputs, probs, training=True):
    out = _masked(inputs, probs)
    try:
        static_training = bool(training)
    except jax.errors.TracerBoolConversionError:
        return jnp.where(training, out, inputs)
    return out if static_training else inputs


# speedup vs baseline: 8.0744x; 1.6818x over previous
import jax
import jax.numpy as jnp
from jax import lax
from jax.experimental import pallas as pl
from jax.experimental.pallas import tpu as pltpu
from jax.experimental.pallas import tpu_sc as plsc

_BATCH = 128
_N = 32768
_L = 16
_NSTEP = _N // _L
_NW = 32
_ROWS_PER_W = _BATCH // _NW


def _f32_to_key(x):
    i = plsc.bitcast(x, jnp.int32)
    u = plsc.bitcast(x, jnp.uint32)
    s = lax.shift_right_arithmetic(i, 31)
    flip = plsc.bitcast(s, jnp.uint32) | jnp.uint32(0x80000000)
    return u ^ flip


def _key_to_f32(k):
    neg = k < jnp.uint32(0x80000000)
    u = jnp.where(neg, ~k, k ^ jnp.uint32(0x80000000))
    return plsc.bitcast(u, jnp.float32)


def _sc_body(x_hbm, probs_hbm, out_hbm, row_v, hist_v, probs_v):
    c = lax.axis_index("c")
    s = lax.axis_index("s")
    wid = s * 2 + c
    pltpu.sync_copy(probs_hbm, probs_v)
    ones = jnp.ones((_L,), jnp.int32)

    def row_body(j, _):
        row = wid * _ROWS_PER_W + j
        pltpu.sync_copy(x_hbm.at[row], row_v)

        p16 = probs_v[pl.ds((row // _L) * _L, _L)]
        sel = lax.iota(jnp.int32, _L) == (row % _L)
        p = jnp.sum(jnp.where(sel, p16, 0.0))
        v = jnp.float32(_N) * p
        vi = v.astype(jnp.int32)
        ceil_v = vi + jnp.where(vi.astype(jnp.float32) < v, 1, 0)
        kv = jnp.clip(ceil_v - 1, 0, _N - 1)

        prefix = jnp.zeros((_L,), jnp.uint32)
        for level in range(4):
            shift = 24 - 8 * level
            for i in range(16):
                hist_v[pl.ds(i * _L, _L)] = jnp.zeros((_L,), jnp.int32)

            def hist_step(i, _, level=level, shift=shift, prefix=prefix):
                x = row_v[pl.ds(i * _L, _L)]
                key = _f32_to_key(x)
                bucket = (key >> jnp.uint32(shift)) & jnp.uint32(0xFF)
                if level == 0:
                    plsc.addupdate_scatter(
                        hist_v, [bucket.astype(jnp.int32)], ones)
                else:
                    m = (key >> jnp.uint32(shift + 8)) == prefix
                    plsc.addupdate_scatter(
                        hist_v, [bucket.astype(jnp.int32)], ones, mask=m)
                return 0

            lax.fori_loop(0, _NSTEP, hist_step, 0, unroll=8)

            carry = jnp.zeros((_L,), jnp.int32)
            bacc = jnp.zeros((_L,), jnp.int32)
            wacc = jnp.zeros((_L,), jnp.int32)
            for i in range(16):
                h = hist_v[pl.ds(i * _L, _L)]
                run = carry + plsc.cumsum(h)
                le = run <= kv
                bacc = bacc + jnp.where(le, 1, 0)
                wacc = wacc + jnp.where(le, h, 0)
                carry = carry + jnp.sum(h)
            bucket = jnp.sum(bacc)
            below = jnp.sum(wacc)
            kv = kv - below
            prefix = (prefix << jnp.uint32(8)) | bucket.astype(jnp.uint32)

        thresh = _key_to_f32(prefix)

        def mask_step(i, _, thresh=thresh):
            x = row_v[pl.ds(i * _L, _L)]
            row_v[pl.ds(i * _L, _L)] = jnp.where(x >= thresh, x, 0.0)
            return 0

        lax.fori_loop(0, _NSTEP, mask_step, 0, unroll=8)
        pltpu.sync_copy(row_v, out_hbm.at[row])
        return 0

    lax.fori_loop(0, _ROWS_PER_W, row_body, 0)


_sc_masking = pl.kernel(
    _sc_body,
    out_type=jax.ShapeDtypeStruct((_BATCH, _N), jnp.float32),
    mesh=plsc.VectorSubcoreMesh(core_axis_name="c", subcore_axis_name="s"),
    compiler_params=pltpu.CompilerParams(needs_layout_passes=False),
    scratch_types=[
        pltpu.VMEM((_N,), jnp.float32),
        pltpu.VMEM((256,), jnp.int32),
        pltpu.VMEM((_BATCH,), jnp.float32),
    ],
)


@jax.jit
def _masked(inputs, probs):
    return _sc_masking(inputs, probs)


def kernel(inputs, probs, training=True):
    out = _masked(inputs, probs)
    try:
        static_training = bool(training)
    except jax.errors.TracerBoolConversionError:
        return jnp.where(training, out, inputs)
    return out if static_training else inputs
